# Initial kernel scaffold; baseline (speedup 1.0000x reference)
import jax
import jax.numpy as jnp
from jax.experimental import pallas as pl


def kernel(x_seq, W_router, W_in, W_rec, b, log_tau, head_W1, head_b1, head_W2, head_b2):
    raise NotImplementedError("write your pallas kernel here")


# speedup vs baseline: 154.9392x; 154.9392x over previous
import jax
import jax.numpy as jnp
from jax.experimental import pallas as pl
from jax.experimental.pallas import tpu as pltpu

_DT = 0.02
_B_TILE = 1024


def _liquid_kernel(xT_ref, wrt_ref, p_ref, w1t_ref, b1_ref, w2t_ref, b2_ref,
                   out_ref):
    S = xT_ref.shape[0]
    Bt = xT_ref.shape[2]
    E = wrt_ref.shape[0]

    wrt = wrt_ref[...]
    P = p_ref[...]
    iota = jax.lax.broadcasted_iota(jnp.int32, (E, Bt), 0)
    big = jnp.int32(E)
    neg_inf = jnp.float32(-jnp.inf)

    def expert_apply(oh, x4, h8):
        pk = jnp.dot(P, oh, preferred_element_type=jnp.float32)
        pre = pk[64:72]
        for d in range(4):
            pre = pre + x4[d:d + 1] * pk[8 * d:8 * d + 8] \
                      + h8[d:d + 1] * pk[32 + 8 * d:40 + 8 * d]
        act = jnp.tanh(pre)
        return h8 + pk[72:80] * (act - h8)

    def step(t, h8):
        x4 = xT_ref[t]
        logits = wrt[:, 0:1] * x4[0:1]
        for d in range(1, 4):
            logits = logits + wrt[:, d:d + 1] * x4[d:d + 1]

        m1 = jnp.max(logits, axis=0, keepdims=True)
        idx1 = jnp.min(jnp.where(logits == m1, iota, big), axis=0,
                       keepdims=True)
        oh1 = (iota == idx1)
        masked = jnp.where(oh1, neg_inf, logits)
        m2 = jnp.max(masked, axis=0, keepdims=True)
        idx2 = jnp.min(jnp.where(masked == m2, iota, big), axis=0,
                       keepdims=True)
        oh2 = (iota == idx2)

        e2 = jnp.exp(m2 - m1)
        g1 = 1.0 / (1.0 + e2)
        g2 = e2 * g1

        hk1 = expert_apply(oh1.astype(jnp.float32), x4, h8)
        hk2 = expert_apply(oh2.astype(jnp.float32), x4, h8)
        return g1 * hk1 + g2 * hk2

    h0 = jnp.zeros((8, Bt), dtype=jnp.float32)
    h8 = jax.lax.fori_loop(0, S, step, h0)

    hidden = jnp.dot(w1t_ref[...], h8, preferred_element_type=jnp.float32)
    hidden = jax.nn.gelu(hidden + b1_ref[...])
    pred = jnp.dot(w2t_ref[...], hidden, preferred_element_type=jnp.float32)
    out_ref[...] = jax.nn.sigmoid(pred + b2_ref[...])


def kernel(x_seq, W_router, W_in, W_rec, b, log_tau, head_W1, head_b1,
           head_W2, head_b2):
    B, S, D = x_seq.shape
    E = W_router.shape[1]
    HID = head_W1.shape[1]
    FP2 = head_W2.shape[1]

    xT = jnp.transpose(x_seq, (1, 2, 0))

    def pack_dd(W):
        Wt = jnp.transpose(W, (1, 2, 0))
        Wt = jnp.pad(Wt, ((0, 0), (0, 4), (0, 0)))
        return Wt.reshape(8 * D, E)

    b8 = jnp.pad(b.T, ((0, 4), (0, 0)))
    dt8 = jnp.pad(_DT * jnp.exp(-log_tau).T, ((0, 4), (0, 0)))
    P = jnp.concatenate([pack_dd(W_in), pack_dd(W_rec), b8, dt8], axis=0)

    wrt = W_router.T
    w1t = jnp.pad(head_W1.T, ((0, 0), (0, 4)))
    b1c = head_b1.reshape(HID, 1)
    OUT_R = 16
    w2t = jnp.pad(head_W2.T, ((0, OUT_R - FP2), (0, 0)))
    b2c = jnp.pad(head_b2, (0, OUT_R - FP2)).reshape(OUT_R, 1)

    bt = min(_B_TILE, B)
    grid = (B // bt,)

    out = pl.pallas_call(
        _liquid_kernel,
        grid=grid,
        in_specs=[
            pl.BlockSpec((S, D, bt), lambda i: (0, 0, i)),
            pl.BlockSpec((E, D), lambda i: (0, 0)),
            pl.BlockSpec((80, E), lambda i: (0, 0)),
            pl.BlockSpec((HID, 8), lambda i: (0, 0)),
            pl.BlockSpec((HID, 1), lambda i: (0, 0)),
            pl.BlockSpec((OUT_R, HID), lambda i: (0, 0)),
            pl.BlockSpec((OUT_R, 1), lambda i: (0, 0)),
        ],
        out_specs=pl.BlockSpec((OUT_R, bt), lambda i: (0, i)),
        out_shape=jax.ShapeDtypeStruct((OUT_R, B), jnp.float32),
        compiler_params=pltpu.CompilerParams(
            dimension_semantics=("parallel",)),
    )(xT, wrt, P, w1t, b1c, w2t, b2c)

    return out[:FP2].T.reshape(B, FP2 // 2, 2)
